# baseline (speedup 1.0000x reference)
import functools
import math

import jax
import jax.numpy as jnp
from jax import lax
from jax.experimental import pallas as pl
from jax.experimental.pallas import tpu as pltpu
from jax.experimental.pallas import tpu_sc as plsc

_SCALE = math.sqrt(512.0)


def _scale_body(t_ref, o_ref):
    o_ref[...] = t_ref[...] * _SCALE


def _scale_table(table):
    return pl.pallas_call(
        _scale_body,
        out_shape=jax.ShapeDtypeStruct(table.shape, table.dtype),
    )(table)


@functools.cache
def _make_gather(BN, S, V, D):
    info = plsc.get_sparse_core_info()
    nc, ns = info.num_cores, info.num_subcores
    nw = nc * ns
    assert BN % nw == 0 and S % 2 == 0
    half = S // 2
    b_per_w = BN // nw
    n_chunks = 2 * b_per_w
    idx_per_w = b_per_w * S
    mesh = plsc.VectorSubcoreMesh(core_axis_name="c", subcore_axis_name="s")

    @functools.partial(
        pl.kernel,
        out_type=jax.ShapeDtypeStruct((BN, S, D), jnp.float32),
        mesh=mesh,
        scratch_types=[
            pltpu.VMEM_SHARED((V, D), jnp.float32),
            pltpu.VMEM((n_chunks, half), jnp.int32),
            pltpu.VMEM((half, D), jnp.float32),
            pltpu.VMEM((half, D), jnp.float32),
            pltpu.SemaphoreType.DMA,
            pltpu.SemaphoreType.DMA,
            pltpu.SemaphoreType.DMA,
            pltpu.SemaphoreType.DMA,
        ],
        compiler_params=pltpu.CompilerParams(use_tc_tiling_on_sc=False),
    )
    def gather(idx_hbm, tbl_hbm, out_hbm, tbl_sh, idx_v, r0, r1,
               g0, g1, o0, o1):
        cid = lax.axis_index("c")
        sid = lax.axis_index("s")
        wid = sid * nc + cid
        rows = (r0, r1)
        gsem = (g0, g1)
        osem = (o0, o1)

        @pl.when(sid == 0)
        def _():
            pltpu.sync_copy(tbl_hbm, tbl_sh)

        pltpu.sync_copy(idx_hbm.at[pl.ds(wid * n_chunks, n_chunks)], idx_v)
        plsc.subcore_barrier()

        def out_slice(i):
            bi = wid * b_per_w + lax.div(i, 2)
            h = lax.rem(i, 2)
            return out_hbm.at[bi, pl.ds(h * half, half)]

        def start_gather(i, b):
            pltpu.async_copy(tbl_sh.at[idx_v.at[i]], rows[b], gsem[b])

        def wait_gather(i, b):
            pltpu.make_async_copy(
                tbl_sh.at[idx_v.at[i]], rows[b], gsem[b]).wait()

        def start_out(i, b):
            pltpu.async_copy(rows[b], out_slice(i), osem[b])

        def wait_out(i, b):
            pltpu.make_async_copy(rows[b], out_slice(i), osem[b]).wait()

        start_gather(0, 0)
        start_gather(1, 1)

        def pair_body(j, carry):
            for b in (0, 1):
                wait_gather(2 * j + b, b)
                start_out(2 * j + b, b)
            for b in (0, 1):
                wait_out(2 * j + b, b)
                start_gather(2 * j + b + 2, b)
            return carry

        lax.fori_loop(0, n_chunks // 2 - 1, pair_body, 0, unroll=False)

        last = n_chunks - 2
        for b in (0, 1):
            wait_gather(last + b, b)
            start_out(last + b, b)
        for b in (0, 1):
            wait_out(last + b, b)

    return gather


def kernel(x, table):
    BN, S = x.shape
    V, D = table.shape
    scaled = _scale_table(table)
    idx2d = x.reshape(BN * 2, S // 2).astype(jnp.int32)
    return _make_gather(BN, S, V, D)(idx2d, scaled)


# speedup vs baseline: 1.1836x; 1.1836x over previous
import functools
import math

import jax
import jax.numpy as jnp
from jax import lax
from jax.experimental import pallas as pl
from jax.experimental.pallas import tpu as pltpu
from jax.experimental.pallas import tpu_sc as plsc

_SCALE = math.sqrt(512.0)
_BB = 128
_NSL = 5
_CW = 200
_NDT = 25


def _scale_body(t_ref, o_ref):
    o_ref[...] = t_ref[...] * _SCALE


def _scale_table(table):
    return pl.pallas_call(
        _scale_body,
        out_shape=jax.ShapeDtypeStruct(table.shape, table.dtype),
    )(table)


@functools.cache
def _make_gather(BN, S, V, D):
    info = plsc.get_sparse_core_info()
    nc, ns, nl = info.num_cores, info.num_subcores, info.num_lanes
    nw = nc * ns
    n_btiles = BN // _BB
    n_blocks = S * n_btiles
    blocks_per_w = n_blocks // nw
    n_steps = blocks_per_w * _NSL
    n_pairs = n_steps // 10
    mesh = plsc.VectorSubcoreMesh(core_axis_name="c", subcore_axis_name="s")

    @functools.partial(
        pl.kernel,
        out_type=jax.ShapeDtypeStruct((S, D // 8, n_btiles, 8, _BB),
                                      jnp.float32),
        mesh=mesh,
        scratch_types=[
            pltpu.VMEM((_BB,), jnp.int32),
            pltpu.VMEM((_BB,), jnp.int32),
            pltpu.VMEM((_BB, _CW), jnp.float32),
            pltpu.VMEM((_BB, _CW), jnp.float32),
            pltpu.VMEM((_NDT, 8, _BB), jnp.float32),
            pltpu.VMEM((_NDT, 8, _BB), jnp.float32),
            pltpu.SemaphoreType.DMA,
            pltpu.SemaphoreType.DMA,
            pltpu.SemaphoreType.DMA,
            pltpu.SemaphoreType.DMA,
        ],
        compiler_params=pltpu.CompilerParams(use_tc_tiling_on_sc=False,
                                             needs_layout_passes=False),
    )
    def gather(xt_hbm, t0, t1, t2, t3, t4, out_hbm,
               idx0, idx1, r0, r1, c0, c1, g0, g1, o0, o1):
        cid = lax.axis_index("c")
        sid = lax.axis_index("s")
        wid = sid * nc + cid
        g_base = wid * blocks_per_w
        tbls = (t0, t1, t2, t3, t4)
        idx = (idx0, idx1)
        rows = (r0, r1)
        chunk = (c0, c1)
        gsem = (g0, g1)
        osem = (o0, o1)

        def load_idx(p, blk):
            g = g_base + blk
            s = lax.div(g, n_btiles)
            bt = lax.rem(g, n_btiles)
            pltpu.sync_copy(xt_hbm.at[s, pl.ds(bt * _BB, _BB)], idx[p])

        def start_gather(k, p, b):
            pltpu.async_copy(tbls[k].at[idx[p]], rows[b], gsem[b])

        def wait_gather(k, p, b):
            pltpu.make_async_copy(tbls[k].at[idx[p]], rows[b],
                                  gsem[b]).wait()

        def out_slice(blk, k):
            g = g_base + blk
            s = lax.div(g, n_btiles)
            bt = lax.rem(g, n_btiles)
            return out_hbm.at[s, pl.ds(k * _NDT, _NDT), bt]

        def start_out(blk, k, b):
            pltpu.async_copy(chunk[b], out_slice(blk, k), osem[b])

        def wait_out(blk, k, b):
            pltpu.make_async_copy(chunk[b], out_slice(blk, k),
                                  osem[b]).wait()

        base_vecs = [
            (lax.iota(jnp.int32, nl) + g * nl) for g in range(8)
        ]

        def transpose_slice(b):
            rb = rows[b]
            cb = chunk[b]

            def w_body(w, carry):
                dtl = lax.div(w, 8)
                ds = lax.rem(w, 8)
                w_vec = jnp.full((nl,), 0, jnp.int32) + w
                for gi in range(8):
                    vals = plsc.load_gather(rb, [base_vecs[gi], w_vec])
                    cb[dtl, ds, pl.ds(gi * nl, nl)] = vals
                return carry

            lax.fori_loop(0, _CW, w_body, 0, unroll=False)

        def emit_step(j, u, first, last):
            k = u % 5
            b = u % 2
            p = u // 5
            blk = 2 * j + u // 5
            if u == 0 and not first:
                load_idx(1, 2 * j + 1)
            if u == 5 and not last:
                load_idx(0, 2 * j + 2)
            wait_gather(k, p, b)
            if not (first and u < 2):
                if u >= 2:
                    wait_out(2 * j + (u - 2) // 5, (u - 2) % 5, b)
                else:
                    wait_out(2 * j - 1, (u + 8) % 5, b)
            transpose_slice(b)
            start_out(blk, k, b)
            if not (last and u >= 8):
                u3 = u + 2
                if u3 <= 9:
                    start_gather(u3 % 5, u3 // 5, b)
                else:
                    start_gather(u3 - 10, 0, b)
            return blk

        load_idx(0, 0)
        load_idx(1, 1)
        start_gather(0, 0, 0)
        start_gather(1, 0, 1)

        for u in range(10):
            emit_step(0, u, True, False)

        def pair_body(j, carry):
            for u in range(10):
                emit_step(j, u, False, False)
            return carry

        lax.fori_loop(1, n_pairs - 1, pair_body, 0, unroll=False)

        for u in range(10):
            emit_step(n_pairs - 1, u, False, True)

        wait_out(2 * (n_pairs - 1) + 1, 3, 0)
        wait_out(2 * (n_pairs - 1) + 1, 4, 1)

    return gather


def kernel(x, table):
    BN, S = x.shape
    V, D = table.shape
    scaled = _scale_table(table)
    tbls = [scaled[:, k * _CW:(k + 1) * _CW] for k in range(_NSL)]
    xt = x.T.astype(jnp.int32)
    o5 = _make_gather(BN, S, V, D)(xt, *tbls)
    return jnp.transpose(o5, (2, 4, 0, 1, 3)).reshape(BN, S, D)
